# baseline (speedup 1.0000x reference)
import functools

import jax
import jax.numpy as jnp
from jax import lax
from jax.experimental import pallas as pl
from jax.experimental.pallas import tpu as pltpu
from jax.experimental.pallas import tpu_sc as plsc

N = 10000
E = 320000
G = 64
IN_CH = 128
HID = 64
FC_W = 128
NCLS = 2

NC, NS = 2, 16
NW = NC * NS
K = 128
EPT = 10240
CPT = EPT // K
EP = EPT * NW
NA = 10240
ZR = NA // NS


def _make_sc_agg(C, NB, stage_src):
    mesh = plsc.VectorSubcoreMesh(
        core_axis_name="c", subcore_axis_name="s", num_cores=NC,
        num_subcores=NS)

    @functools.partial(
        pl.kernel,
        out_type=jax.ShapeDtypeStruct((NC, NA, C), jnp.float32),
        mesh=mesh,
        scratch_types=(
            ([pltpu.VMEM((CPT, K), jnp.int32)] if stage_src
             else [pltpu.VMEM((K,), jnp.int32) for _ in range(NB)])
            + [pltpu.VMEM((K,), jnp.int32) for _ in range(NB)]
            + [pltpu.VMEM((K, C), jnp.float32) for _ in range(NB)]
            + [pltpu.VMEM_SHARED((NA, C), jnp.float32)]
            + [pltpu.SemaphoreType.DMA for _ in range(3 * NB)]
        ),
        compiler_params=pltpu.CompilerParams(use_tc_tiling_on_sc=False),
    )
    def agg(table, src_in, zrows, dstf, out, *rest):
        ns = 1 if stage_src else NB
        src_r = rest[:ns]
        dst_c = rest[ns:ns + NB]
        rows = rest[ns + NB:ns + 2 * NB]
        acc = rest[ns + 2 * NB]
        sems = rest[ns + 2 * NB + 1:]
        sem_g = sems[:NB]
        sem_d = sems[NB:2 * NB]
        sem_i = sems[2 * NB:3 * NB]
        cid = lax.axis_index("c")
        sid = lax.axis_index("s")
        wid = sid * NC + cid
        pltpu.sync_copy(zrows, acc.at[pl.ds(sid * ZR, ZR)])
        if stage_src:
            pltpu.sync_copy(src_in.at[pl.ds(wid * CPT, CPT)], src_r[0])
        plsc.subcore_barrier()

        def body(i, carry):
            jbase = i * NB
            idx_loads = []
            for b in range(NB):
                j = jbase + b
                dd = pltpu.async_copy(
                    dstf.at[pl.ds(wid * EPT + j * K, K)], dst_c[b], sem_d[b])
                sl = None
                if not stage_src:
                    sl = pltpu.async_copy(
                        src_in.at[pl.ds(wid * EPT + j * K, K)], src_r[b],
                        sem_i[b])
                idx_loads.append((dd, sl))
            gathers = []
            for b in range(NB):
                j = jbase + b
                if stage_src:
                    sidx = src_r[0].at[j]
                else:
                    idx_loads[b][1].wait()
                    sidx = src_r[b]
                gathers.append(pltpu.async_copy(
                    table.at[sidx], rows[b], sem_g[b]))
            for b in range(NB):
                gathers[b].wait()
                idx_loads[b][0].wait()
                pltpu.sync_copy(rows[b], acc.at[dst_c[b]], add=True)
            return carry

        lax.fori_loop(0, CPT // NB, body, 0)
        plsc.subcore_barrier()
        pltpu.sync_copy(acc.at[pl.ds(sid * ZR, ZR)],
                        out.at[cid, pl.ds(sid * ZR, ZR)])

    return agg


_SC_AGG_CACHE = {}


def _sc_agg(C, NB, stage_src):
    key = (C, NB, stage_src)
    if key not in _SC_AGG_CACHE:
        _SC_AGG_CACHE[key] = _make_sc_agg(C, NB, stage_src)
    return _SC_AGG_CACHE[key]


def _l0_body(p_ref, x_ref, mwl, mwr, vwl, vwr, mb, vb, h_out, inv_out):
    p = (p_ref[0] + p_ref[1])[:N]
    inv = 1.0 / jnp.maximum(p[:, IN_CH:IN_CH + 1], 1.0)
    agg = p[:, :IN_CH] * inv
    x = x_ref[...]
    m = jnp.maximum(agg @ mwl[...] + mb[...] + x @ mwr[...], 0.0)
    v = jnp.maximum(agg @ vwl[...] + vb[...] + x @ vwr[...], 0.0)
    h_out[...] = jnp.concatenate([m, v], axis=1)
    inv_out[...] = inv


def _layer_body(p_ref, h_ref, inv_ref, mwl, mwr, vwl, vwr, mb, vb, h_out):
    p = (p_ref[0] + p_ref[1])[:N]
    inv = inv_ref[...]
    h = h_ref[...]
    m = jnp.maximum(p[:, :HID] * inv @ mwl[...] + mb[...]
                    + h[:, :HID] @ mwr[...], 0.0)
    v = jnp.maximum(p[:, HID:] * inv @ vwl[...] + vb[...]
                    + h[:, HID:] @ vwr[...], 0.0)
    h_out[...] = jnp.concatenate([m, v], axis=1)


def _head_body(h_ref, eps_ref, b_ref, f1w, f1b, f2w, f2b, out_ref):
    h = h_ref[...]
    z = h[:, :HID] + eps_ref[...] * jnp.exp(0.5 * h[:, HID:])
    gid = lax.broadcasted_iota(jnp.int32, (N, G), 1)
    oh = (b_ref[...] == gid).astype(jnp.float32)
    zsum = lax.dot_general(oh, z, (((0,), (0,)), ((), ())))
    cnt = jnp.sum(oh, axis=0)[:, None]
    zp = zsum / jnp.maximum(cnt, 1.0)
    a = jnp.maximum(zp @ f1w[...] + f1b[...], 0.0)
    logits = a @ f2w[...] + f2b[...]
    mx = jnp.max(logits, axis=1, keepdims=True)
    lse = mx + jnp.log(jnp.sum(jnp.exp(logits - mx), axis=1, keepdims=True))
    out_ref[...] = logits - lse


_EPS_CACHE = None


def _eps():
    global _EPS_CACHE
    if _EPS_CACHE is None:
        _EPS_CACHE = jax.random.normal(
            jax.random.key(42), (N, HID), jnp.float32)
    return _EPS_CACHE


def kernel(x, edge_index, edge_attr, batch, params):
    f32 = jnp.float32
    src = edge_index[0]
    dst = edge_index[1]
    srcf = jnp.concatenate([src, jnp.zeros((EP - E,), jnp.int32)])
    srcp = srcf.reshape(EP // K, K)
    dstf = jnp.concatenate([dst, jnp.full((EP - E,), N, jnp.int32)])
    z144 = jnp.zeros((ZR, IN_CH + 16), f32)
    z128 = jnp.zeros((ZR, 2 * HID), f32)

    table0 = jnp.concatenate(
        [x, jnp.ones((N, 1), f32), jnp.zeros((N, 15), f32)], axis=1)
    p0 = _sc_agg(IN_CH + 16, 2, False)(table0, srcf, z144, dstf)

    b1 = lambda name: params[name].reshape(1, HID)
    h1, inv = pl.pallas_call(
        _l0_body,
        out_shape=(jax.ShapeDtypeStruct((N, 2 * HID), f32),
                   jax.ShapeDtypeStruct((N, 1), f32)),
    )(p0, x, params['mW_l0'], params['mW_r0'], params['vW_l0'],
      params['vW_r0'], b1('mb_l0'), b1('vb_l0'))

    h = h1
    for i in (1, 2):
        p = _sc_agg(2 * HID, 2, True)(h, srcp, z128, dstf)
        h = pl.pallas_call(
            _layer_body,
            out_shape=jax.ShapeDtypeStruct((N, 2 * HID), f32),
        )(p, h, inv, params[f'mW_l{i}'], params[f'mW_r{i}'],
          params[f'vW_l{i}'], params[f'vW_r{i}'],
          b1(f'mb_l{i}'), b1(f'vb_l{i}'))

    logp = pl.pallas_call(
        _head_body,
        out_shape=jax.ShapeDtypeStruct((G, NCLS), f32),
    )(h, _eps(), batch.reshape(N, 1), params['fc1_W'],
      params['fc1_b'].reshape(1, FC_W), params['fc2_W'],
      params['fc2_b'].reshape(1, NCLS))

    return (logp, h[:, :HID], h[:, HID:])


# speedup vs baseline: 1.0027x; 1.0027x over previous
import functools

import jax
import jax.numpy as jnp
from jax import lax
from jax.experimental import pallas as pl
from jax.experimental.pallas import tpu as pltpu
from jax.experimental.pallas import tpu_sc as plsc

N = 10000
E = 320000
G = 64
IN_CH = 128
HID = 64
FC_W = 128
NCLS = 2

NC, NS = 2, 16
NW = NC * NS
K = 128
EPT = 10240
CPT = EPT // K
EP = EPT * NW
NA = 10240
ZR = NA // NS


def _make_sc_agg(C, NB, stage_src):
    mesh = plsc.VectorSubcoreMesh(
        core_axis_name="c", subcore_axis_name="s", num_cores=NC,
        num_subcores=NS)

    @functools.partial(
        pl.kernel,
        out_type=jax.ShapeDtypeStruct((NC, NA, C), jnp.float32),
        mesh=mesh,
        scratch_types=(
            ([pltpu.VMEM((CPT, K), jnp.int32)] if stage_src
             else [pltpu.VMEM((K,), jnp.int32) for _ in range(NB)])
            + [pltpu.VMEM((K,), jnp.int32) for _ in range(NB)]
            + [pltpu.VMEM((K, C), jnp.float32) for _ in range(NB)]
            + [pltpu.VMEM_SHARED((NA, C), jnp.float32)]
            + [pltpu.SemaphoreType.DMA for _ in range(4 * NB)]
        ),
        compiler_params=pltpu.CompilerParams(use_tc_tiling_on_sc=False),
    )
    def agg(table, src_in, zrows, dstf, out, *rest):
        ns = 1 if stage_src else NB
        src_r = rest[:ns]
        dst_c = rest[ns:ns + NB]
        rows = rest[ns + NB:ns + 2 * NB]
        acc = rest[ns + 2 * NB]
        sems = rest[ns + 2 * NB + 1:]
        sem_g = sems[:NB]
        sem_d = sems[NB:2 * NB]
        sem_i = sems[2 * NB:3 * NB]
        sem_s = sems[3 * NB:4 * NB]
        cid = lax.axis_index("c")
        sid = lax.axis_index("s")
        wid = sid * NC + cid
        pltpu.sync_copy(zrows, acc.at[pl.ds(sid * ZR, ZR)])
        if stage_src:
            pltpu.sync_copy(src_in.at[pl.ds(wid * CPT, CPT)], src_r[0])
        plsc.subcore_barrier()

        def body(i, carry):
            jbase = i * NB
            idx_loads = []
            for b in range(NB):
                j = jbase + b
                dd = pltpu.async_copy(
                    dstf.at[pl.ds(wid * EPT + j * K, K)], dst_c[b], sem_d[b])
                sl = None
                if not stage_src:
                    sl = pltpu.async_copy(
                        src_in.at[pl.ds(wid * EPT + j * K, K)], src_r[b],
                        sem_i[b])
                idx_loads.append((dd, sl))
            gathers = []
            for b in range(NB):
                j = jbase + b
                if stage_src:
                    sidx = src_r[0].at[j]
                else:
                    idx_loads[b][1].wait()
                    sidx = src_r[b]
                gathers.append(pltpu.async_copy(
                    table.at[sidx], rows[b], sem_g[b]))
            scats = []
            for b in range(NB):
                gathers[b].wait()
                idx_loads[b][0].wait()
                scats.append(pltpu.async_copy(
                    rows[b], acc.at[dst_c[b]], sem_s[b], add=True))
            for s in scats:
                s.wait()
            return carry

        lax.fori_loop(0, CPT // NB, body, 0)
        plsc.subcore_barrier()
        pltpu.sync_copy(acc.at[pl.ds(sid * ZR, ZR)],
                        out.at[cid, pl.ds(sid * ZR, ZR)])

    return agg


_SC_AGG_CACHE = {}


def _sc_agg(C, NB, stage_src):
    key = (C, NB, stage_src)
    if key not in _SC_AGG_CACHE:
        _SC_AGG_CACHE[key] = _make_sc_agg(C, NB, stage_src)
    return _SC_AGG_CACHE[key]


def _l0_body(p_ref, x_ref, mwl, mwr, vwl, vwr, mb, vb, h_out, inv_out):
    p = (p_ref[0] + p_ref[1])[:N]
    inv = 1.0 / jnp.maximum(p[:, IN_CH:IN_CH + 1], 1.0)
    agg = p[:, :IN_CH] * inv
    x = x_ref[...]
    m = jnp.maximum(agg @ mwl[...] + mb[...] + x @ mwr[...], 0.0)
    v = jnp.maximum(agg @ vwl[...] + vb[...] + x @ vwr[...], 0.0)
    h_out[...] = jnp.concatenate([m, v], axis=1)
    inv_out[...] = inv


def _layer_body(p_ref, h_ref, inv_ref, mwl, mwr, vwl, vwr, mb, vb, h_out):
    p = (p_ref[0] + p_ref[1])[:N]
    inv = inv_ref[...]
    h = h_ref[...]
    m = jnp.maximum(p[:, :HID] * inv @ mwl[...] + mb[...]
                    + h[:, :HID] @ mwr[...], 0.0)
    v = jnp.maximum(p[:, HID:] * inv @ vwl[...] + vb[...]
                    + h[:, HID:] @ vwr[...], 0.0)
    h_out[...] = jnp.concatenate([m, v], axis=1)


def _head_body(h_ref, eps_ref, b_ref, f1w, f1b, f2w, f2b, out_ref):
    h = h_ref[...]
    z = h[:, :HID] + eps_ref[...] * jnp.exp(0.5 * h[:, HID:])
    gid = lax.broadcasted_iota(jnp.int32, (N, G), 1)
    oh = (b_ref[...] == gid).astype(jnp.float32)
    zsum = lax.dot_general(oh, z, (((0,), (0,)), ((), ())))
    cnt = jnp.sum(oh, axis=0)[:, None]
    zp = zsum / jnp.maximum(cnt, 1.0)
    a = jnp.maximum(zp @ f1w[...] + f1b[...], 0.0)
    logits = a @ f2w[...] + f2b[...]
    mx = jnp.max(logits, axis=1, keepdims=True)
    lse = mx + jnp.log(jnp.sum(jnp.exp(logits - mx), axis=1, keepdims=True))
    out_ref[...] = logits - lse


_EPS_CACHE = None


def _eps():
    global _EPS_CACHE
    if _EPS_CACHE is None:
        _EPS_CACHE = jax.random.normal(
            jax.random.key(42), (N, HID), jnp.float32)
    return _EPS_CACHE


def kernel(x, edge_index, edge_attr, batch, params):
    f32 = jnp.float32
    src = edge_index[0]
    dst = edge_index[1]
    srcf = jnp.concatenate([src, jnp.zeros((EP - E,), jnp.int32)])
    srcp = srcf.reshape(EP // K, K)
    dstf = jnp.concatenate([dst, jnp.full((EP - E,), N, jnp.int32)])
    z144 = jnp.zeros((ZR, IN_CH + 16), f32)
    z128 = jnp.zeros((ZR, 2 * HID), f32)

    table0 = jnp.concatenate(
        [x, jnp.ones((N, 1), f32), jnp.zeros((N, 15), f32)], axis=1)
    p0 = _sc_agg(IN_CH + 16, 2, False)(table0, srcf, z144, dstf)

    b1 = lambda name: params[name].reshape(1, HID)
    h1, inv = pl.pallas_call(
        _l0_body,
        out_shape=(jax.ShapeDtypeStruct((N, 2 * HID), f32),
                   jax.ShapeDtypeStruct((N, 1), f32)),
    )(p0, x, params['mW_l0'], params['mW_r0'], params['vW_l0'],
      params['vW_r0'], b1('mb_l0'), b1('vb_l0'))

    h = h1
    for i in (1, 2):
        p = _sc_agg(2 * HID, 2, True)(h, srcp, z128, dstf)
        h = pl.pallas_call(
            _layer_body,
            out_shape=jax.ShapeDtypeStruct((N, 2 * HID), f32),
        )(p, h, inv, params[f'mW_l{i}'], params[f'mW_r{i}'],
          params[f'vW_l{i}'], params[f'vW_r{i}'],
          b1(f'mb_l{i}'), b1(f'vb_l{i}'))

    logp = pl.pallas_call(
        _head_body,
        out_shape=jax.ShapeDtypeStruct((G, NCLS), f32),
    )(h, _eps(), batch.reshape(N, 1), params['fc1_W'],
      params['fc1_b'].reshape(1, FC_W), params['fc2_W'],
      params['fc2_b'].reshape(1, NCLS))

    return (logp, h[:, :HID], h[:, HID:])


# speedup vs baseline: 1.0375x; 1.0348x over previous
import functools

import jax
import jax.numpy as jnp
from jax import lax
from jax.experimental import pallas as pl
from jax.experimental.pallas import tpu as pltpu
from jax.experimental.pallas import tpu_sc as plsc

N = 10000
E = 320000
G = 64
IN_CH = 128
HID = 64
FC_W = 128
NCLS = 2

NC, NS = 2, 16
NW = NC * NS
K = 128
EPT = 10240
CPT = EPT // K
EP = EPT * NW
NA = 10240
ZR = NA // NS
CB = 4


def _make_sc_agg(C, NB, stage_src):
    mesh = plsc.VectorSubcoreMesh(
        core_axis_name="c", subcore_axis_name="s", num_cores=NC,
        num_subcores=NS)

    @functools.partial(
        pl.kernel,
        out_type=jax.ShapeDtypeStruct((NC, NA, C), jnp.float32),
        mesh=mesh,
        scratch_types=(
            ([pltpu.VMEM((CPT, K), jnp.int32)] if stage_src
             else [pltpu.VMEM((K,), jnp.int32) for _ in range(CB)])
            + [pltpu.VMEM((K,), jnp.int32) for _ in range(CB)]
            + [pltpu.VMEM((K, C), jnp.float32) for _ in range(2)]
            + [pltpu.VMEM_SHARED((NA, C), jnp.float32)]
            + [pltpu.SemaphoreType.DMA for _ in range(4 + 2 * CB)]
        ),
        compiler_params=pltpu.CompilerParams(use_tc_tiling_on_sc=False),
    )
    def agg(table, src_in, zrows, dstf, out, *rest):
        ns = 1 if stage_src else CB
        src_r = rest[:ns]
        dst_c = rest[ns:ns + CB]
        rows = rest[ns + CB:ns + CB + 2]
        acc = rest[ns + CB + 2]
        sems = rest[ns + CB + 3:]
        sem_g = sems[0:2]
        sem_s = sems[2:4]
        sem_d = sems[4:4 + CB]
        sem_i = sems[4 + CB:4 + 2 * CB]
        cid = lax.axis_index("c")
        sid = lax.axis_index("s")
        wid = sid * NC + cid
        pltpu.sync_copy(zrows, acc.at[pl.ds(sid * ZR, ZR)])
        if stage_src:
            pltpu.sync_copy(src_in.at[pl.ds(wid * CPT, CPT)], src_r[0])
        plsc.subcore_barrier()

        def body(i, carry):
            q0 = i * CB
            idx = []
            for t in range(CB):
                base = wid * EPT + (q0 + t) * K
                dd = pltpu.async_copy(
                    dstf.at[pl.ds(base, K)], dst_c[t], sem_d[t])
                sl = None
                if not stage_src:
                    sl = pltpu.async_copy(
                        src_in.at[pl.ds(base, K)], src_r[t], sem_i[t])
                idx.append((dd, sl))
            g = [None] * CB
            s = [None] * CB
            for t in range(CB):
                if t >= 2:
                    s[t - 2].wait()
                idx[t][0].wait()
                if stage_src:
                    sidx = src_r[0].at[q0 + t]
                else:
                    idx[t][1].wait()
                    sidx = src_r[t]
                g[t] = pltpu.async_copy(table.at[sidx], rows[t % 2],
                                        sem_g[t % 2])
                if t >= 1:
                    g[t - 1].wait()
                    s[t - 1] = pltpu.async_copy(
                        rows[(t - 1) % 2], acc.at[dst_c[t - 1]],
                        sem_s[(t - 1) % 2], add=True)
            g[CB - 1].wait()
            s[CB - 1] = pltpu.async_copy(
                rows[(CB - 1) % 2], acc.at[dst_c[CB - 1]],
                sem_s[(CB - 1) % 2], add=True)
            s[CB - 2].wait()
            s[CB - 1].wait()
            return carry

        lax.fori_loop(0, CPT // CB, body, 0)
        plsc.subcore_barrier()
        pltpu.sync_copy(acc.at[pl.ds(sid * ZR, ZR)],
                        out.at[cid, pl.ds(sid * ZR, ZR)])

    return agg


_SC_AGG_CACHE = {}


def _sc_agg(C, NB, stage_src):
    key = (C, NB, stage_src)
    if key not in _SC_AGG_CACHE:
        _SC_AGG_CACHE[key] = _make_sc_agg(C, NB, stage_src)
    return _SC_AGG_CACHE[key]


def _l0_body(p_ref, x_ref, mwl, mwr, vwl, vwr, mb, vb, h_out, inv_out):
    p = (p_ref[0] + p_ref[1])[:N]
    inv = 1.0 / jnp.maximum(p[:, IN_CH:IN_CH + 1], 1.0)
    agg = p[:, :IN_CH] * inv
    x = x_ref[...]
    m = jnp.maximum(agg @ mwl[...] + mb[...] + x @ mwr[...], 0.0)
    v = jnp.maximum(agg @ vwl[...] + vb[...] + x @ vwr[...], 0.0)
    h_out[...] = jnp.concatenate([m, v], axis=1)
    inv_out[...] = inv


def _layer_body(p_ref, h_ref, inv_ref, mwl, mwr, vwl, vwr, mb, vb, h_out):
    p = (p_ref[0] + p_ref[1])[:N]
    inv = inv_ref[...]
    h = h_ref[...]
    m = jnp.maximum(p[:, :HID] * inv @ mwl[...] + mb[...]
                    + h[:, :HID] @ mwr[...], 0.0)
    v = jnp.maximum(p[:, HID:] * inv @ vwl[...] + vb[...]
                    + h[:, HID:] @ vwr[...], 0.0)
    h_out[...] = jnp.concatenate([m, v], axis=1)


def _head_body(h_ref, eps_ref, b_ref, f1w, f1b, f2w, f2b, out_ref):
    h = h_ref[...]
    z = h[:, :HID] + eps_ref[...] * jnp.exp(0.5 * h[:, HID:])
    gid = lax.broadcasted_iota(jnp.int32, (N, G), 1)
    oh = (b_ref[...] == gid).astype(jnp.float32)
    zsum = lax.dot_general(oh, z, (((0,), (0,)), ((), ())))
    cnt = jnp.sum(oh, axis=0)[:, None]
    zp = zsum / jnp.maximum(cnt, 1.0)
    a = jnp.maximum(zp @ f1w[...] + f1b[...], 0.0)
    logits = a @ f2w[...] + f2b[...]
    mx = jnp.max(logits, axis=1, keepdims=True)
    lse = mx + jnp.log(jnp.sum(jnp.exp(logits - mx), axis=1, keepdims=True))
    out_ref[...] = logits - lse


_EPS_CACHE = None


def _eps():
    global _EPS_CACHE
    if _EPS_CACHE is None:
        _EPS_CACHE = jax.random.normal(
            jax.random.key(42), (N, HID), jnp.float32)
    return _EPS_CACHE


def kernel(x, edge_index, edge_attr, batch, params):
    f32 = jnp.float32
    src = edge_index[0]
    dst = edge_index[1]
    srcf = jnp.concatenate([src, jnp.zeros((EP - E,), jnp.int32)])
    srcp = srcf.reshape(EP // K, K)
    dstf = jnp.concatenate([dst, jnp.full((EP - E,), N, jnp.int32)])
    z144 = jnp.zeros((ZR, IN_CH + 16), f32)
    z128 = jnp.zeros((ZR, 2 * HID), f32)

    table0 = jnp.concatenate(
        [x, jnp.ones((N, 1), f32), jnp.zeros((N, 15), f32)], axis=1)
    p0 = _sc_agg(IN_CH + 16, 2, False)(table0, srcf, z144, dstf)

    b1 = lambda name: params[name].reshape(1, HID)
    h1, inv = pl.pallas_call(
        _l0_body,
        out_shape=(jax.ShapeDtypeStruct((N, 2 * HID), f32),
                   jax.ShapeDtypeStruct((N, 1), f32)),
    )(p0, x, params['mW_l0'], params['mW_r0'], params['vW_l0'],
      params['vW_r0'], b1('mb_l0'), b1('vb_l0'))

    h = h1
    for i in (1, 2):
        p = _sc_agg(2 * HID, 2, True)(h, srcp, z128, dstf)
        h = pl.pallas_call(
            _layer_body,
            out_shape=jax.ShapeDtypeStruct((N, 2 * HID), f32),
        )(p, h, inv, params[f'mW_l{i}'], params[f'mW_r{i}'],
          params[f'vW_l{i}'], params[f'vW_r{i}'],
          b1(f'mb_l{i}'), b1(f'vb_l{i}'))

    logp = pl.pallas_call(
        _head_body,
        out_shape=jax.ShapeDtypeStruct((G, NCLS), f32),
    )(h, _eps(), batch.reshape(N, 1), params['fc1_W'],
      params['fc1_b'].reshape(1, FC_W), params['fc2_W'],
      params['fc2_b'].reshape(1, NCLS))

    return (logp, h[:, :HID], h[:, HID:])


# speedup vs baseline: 1.0539x; 1.0158x over previous
import functools

import jax
import jax.numpy as jnp
from jax import lax
from jax.experimental import pallas as pl
from jax.experimental.pallas import tpu as pltpu
from jax.experimental.pallas import tpu_sc as plsc

N = 10000
E = 320000
G = 64
IN_CH = 128
HID = 64
FC_W = 128
NCLS = 2

NC, NS = 2, 16
NW = NC * NS
K = 128
EPT = 10240
CPT = EPT // K
EP = EPT * NW
NA = 10240
ZR = NA // NS
CB = 5


def _make_sc_agg(C, NB, stage_src):
    mesh = plsc.VectorSubcoreMesh(
        core_axis_name="c", subcore_axis_name="s", num_cores=NC,
        num_subcores=NS)

    @functools.partial(
        pl.kernel,
        out_type=jax.ShapeDtypeStruct((NC, NA, C), jnp.float32),
        mesh=mesh,
        scratch_types=(
            ([pltpu.VMEM((CPT, K), jnp.int32)] if stage_src
             else [pltpu.VMEM((K,), jnp.int32) for _ in range(CB)])
            + [pltpu.VMEM((K,), jnp.int32) for _ in range(CB)]
            + [pltpu.VMEM((K, C), jnp.float32) for _ in range(2)]
            + [pltpu.VMEM_SHARED((NA, C), jnp.float32)]
            + [pltpu.SemaphoreType.DMA for _ in range(4 + 2 * CB)]
        ),
        compiler_params=pltpu.CompilerParams(use_tc_tiling_on_sc=False),
    )
    def agg(table, src_in, zrows, dstf, out, *rest):
        ns = 1 if stage_src else CB
        src_r = rest[:ns]
        dst_c = rest[ns:ns + CB]
        rows = rest[ns + CB:ns + CB + 2]
        acc = rest[ns + CB + 2]
        sems = rest[ns + CB + 3:]
        sem_g = sems[0:2]
        sem_s = sems[2:4]
        sem_d = sems[4:4 + CB]
        sem_i = sems[4 + CB:4 + 2 * CB]
        cid = lax.axis_index("c")
        sid = lax.axis_index("s")
        wid = sid * NC + cid
        pltpu.sync_copy(zrows, acc.at[pl.ds(sid * ZR, ZR)])
        if stage_src:
            pltpu.sync_copy(src_in.at[pl.ds(wid * CPT, CPT)], src_r[0])
        plsc.subcore_barrier()

        def body(i, carry):
            q0 = i * CB
            idx = []
            for t in range(CB):
                base = wid * EPT + (q0 + t) * K
                dd = pltpu.async_copy(
                    dstf.at[pl.ds(base, K)], dst_c[t], sem_d[t])
                sl = None
                if not stage_src:
                    sl = pltpu.async_copy(
                        src_in.at[pl.ds(base, K)], src_r[t], sem_i[t])
                idx.append((dd, sl))
            g = [None] * CB
            s = [None] * CB
            for t in range(CB):
                if t >= 2:
                    s[t - 2].wait()
                if stage_src:
                    sidx = src_r[0].at[q0 + t]
                else:
                    idx[t][1].wait()
                    sidx = src_r[t]
                g[t] = pltpu.async_copy(table.at[sidx], rows[t % 2],
                                        sem_g[t % 2])
                if t >= 1:
                    g[t - 1].wait()
                    idx[t - 1][0].wait()
                    s[t - 1] = pltpu.async_copy(
                        rows[(t - 1) % 2], acc.at[dst_c[t - 1]],
                        sem_s[(t - 1) % 2], add=True)
            g[CB - 1].wait()
            idx[CB - 1][0].wait()
            s[CB - 1] = pltpu.async_copy(
                rows[(CB - 1) % 2], acc.at[dst_c[CB - 1]],
                sem_s[(CB - 1) % 2], add=True)
            s[CB - 2].wait()
            s[CB - 1].wait()
            return carry

        lax.fori_loop(0, CPT // CB, body, 0)
        plsc.subcore_barrier()
        pltpu.sync_copy(acc.at[pl.ds(sid * ZR, ZR)],
                        out.at[cid, pl.ds(sid * ZR, ZR)])

    return agg


_SC_AGG_CACHE = {}


def _sc_agg(C, NB, stage_src):
    key = (C, NB, stage_src)
    if key not in _SC_AGG_CACHE:
        _SC_AGG_CACHE[key] = _make_sc_agg(C, NB, stage_src)
    return _SC_AGG_CACHE[key]


def _l0_body(p_ref, x_ref, mwl, mwr, vwl, vwr, mb, vb, h_out, inv_out):
    p = (p_ref[0] + p_ref[1])[:N]
    inv = 1.0 / jnp.maximum(p[:, IN_CH:IN_CH + 1], 1.0)
    agg = p[:, :IN_CH] * inv
    x = x_ref[...]
    m = jnp.maximum(agg @ mwl[...] + mb[...] + x @ mwr[...], 0.0)
    v = jnp.maximum(agg @ vwl[...] + vb[...] + x @ vwr[...], 0.0)
    h_out[...] = jnp.concatenate([m, v], axis=1)
    inv_out[...] = inv


def _layer_body(p_ref, h_ref, inv_ref, mwl, mwr, vwl, vwr, mb, vb, h_out):
    p = (p_ref[0] + p_ref[1])[:N]
    inv = inv_ref[...]
    h = h_ref[...]
    m = jnp.maximum(p[:, :HID] * inv @ mwl[...] + mb[...]
                    + h[:, :HID] @ mwr[...], 0.0)
    v = jnp.maximum(p[:, HID:] * inv @ vwl[...] + vb[...]
                    + h[:, HID:] @ vwr[...], 0.0)
    h_out[...] = jnp.concatenate([m, v], axis=1)


def _head_body(h_ref, eps_ref, b_ref, f1w, f1b, f2w, f2b, out_ref):
    h = h_ref[...]
    z = h[:, :HID] + eps_ref[...] * jnp.exp(0.5 * h[:, HID:])
    gid = lax.broadcasted_iota(jnp.int32, (N, G), 1)
    oh = (b_ref[...] == gid).astype(jnp.float32)
    zsum = lax.dot_general(oh, z, (((0,), (0,)), ((), ())))
    cnt = jnp.sum(oh, axis=0)[:, None]
    zp = zsum / jnp.maximum(cnt, 1.0)
    a = jnp.maximum(zp @ f1w[...] + f1b[...], 0.0)
    logits = a @ f2w[...] + f2b[...]
    mx = jnp.max(logits, axis=1, keepdims=True)
    lse = mx + jnp.log(jnp.sum(jnp.exp(logits - mx), axis=1, keepdims=True))
    out_ref[...] = logits - lse


_EPS_CACHE = None


def _eps():
    global _EPS_CACHE
    if _EPS_CACHE is None:
        _EPS_CACHE = jax.random.normal(
            jax.random.key(42), (N, HID), jnp.float32)
    return _EPS_CACHE


def kernel(x, edge_index, edge_attr, batch, params):
    f32 = jnp.float32
    src = edge_index[0]
    dst = edge_index[1]
    srcf = jnp.concatenate([src, jnp.zeros((EP - E,), jnp.int32)])
    srcp = srcf.reshape(EP // K, K)
    dstf = jnp.concatenate([dst, jnp.full((EP - E,), N, jnp.int32)])
    z144 = jnp.zeros((ZR, IN_CH + 16), f32)
    z128 = jnp.zeros((ZR, 2 * HID), f32)

    table0 = jnp.concatenate(
        [x, jnp.ones((N, 1), f32), jnp.zeros((N, 15), f32)], axis=1)
    p0 = _sc_agg(IN_CH + 16, 2, False)(table0, srcf, z144, dstf)

    b1 = lambda name: params[name].reshape(1, HID)
    h1, inv = pl.pallas_call(
        _l0_body,
        out_shape=(jax.ShapeDtypeStruct((N, 2 * HID), f32),
                   jax.ShapeDtypeStruct((N, 1), f32)),
    )(p0, x, params['mW_l0'], params['mW_r0'], params['vW_l0'],
      params['vW_r0'], b1('mb_l0'), b1('vb_l0'))

    h = h1
    for i in (1, 2):
        p = _sc_agg(2 * HID, 2, True)(h, srcp, z128, dstf)
        h = pl.pallas_call(
            _layer_body,
            out_shape=jax.ShapeDtypeStruct((N, 2 * HID), f32),
        )(p, h, inv, params[f'mW_l{i}'], params[f'mW_r{i}'],
          params[f'vW_l{i}'], params[f'vW_r{i}'],
          b1(f'mb_l{i}'), b1(f'vb_l{i}'))

    logp = pl.pallas_call(
        _head_body,
        out_shape=jax.ShapeDtypeStruct((G, NCLS), f32),
    )(h, _eps(), batch.reshape(N, 1), params['fc1_W'],
      params['fc1_b'].reshape(1, FC_W), params['fc2_W'],
      params['fc2_b'].reshape(1, NCLS))

    return (logp, h[:, :HID], h[:, HID:])


# speedup vs baseline: 1.0591x; 1.0049x over previous
import functools

import jax
import jax.numpy as jnp
from jax import lax
from jax.experimental import pallas as pl
from jax.experimental.pallas import tpu as pltpu
from jax.experimental.pallas import tpu_sc as plsc

N = 10000
E = 320000
G = 64
IN_CH = 128
HID = 64
FC_W = 128
NCLS = 2

NC, NS = 2, 16
NW = NC * NS
K = 128
EPT = 10240
CPT = EPT // K
EP = EPT * NW
NA = 10240
ZR = NA // NS


def _make_sc_agg(C, CB, stage_src):
    mesh = plsc.VectorSubcoreMesh(
        core_axis_name="c", subcore_axis_name="s", num_cores=NC,
        num_subcores=NS)

    @functools.partial(
        pl.kernel,
        out_type=jax.ShapeDtypeStruct((NC, NA, C), jnp.float32),
        mesh=mesh,
        scratch_types=(
            ([pltpu.VMEM((CPT, K), jnp.int32)] if stage_src
             else [pltpu.VMEM((K,), jnp.int32) for _ in range(CB)])
            + [pltpu.VMEM((K,), jnp.int32) for _ in range(CB)]
            + [pltpu.VMEM((K, C), jnp.float32) for _ in range(2)]
            + [pltpu.VMEM_SHARED((NA, C), jnp.float32)]
            + [pltpu.SemaphoreType.DMA for _ in range(4 + 2 * CB)]
        ),
        compiler_params=pltpu.CompilerParams(use_tc_tiling_on_sc=False),
    )
    def agg(table, src_in, zrows, dstf, out, *rest):
        ns = 1 if stage_src else CB
        src_r = rest[:ns]
        dst_c = rest[ns:ns + CB]
        rows = rest[ns + CB:ns + CB + 2]
        acc = rest[ns + CB + 2]
        sems = rest[ns + CB + 3:]
        sem_g = sems[0:2]
        sem_s = sems[2:4]
        sem_d = sems[4:4 + CB]
        sem_i = sems[4 + CB:4 + 2 * CB]
        cid = lax.axis_index("c")
        sid = lax.axis_index("s")
        wid = sid * NC + cid
        pltpu.sync_copy(zrows, acc.at[pl.ds(sid * ZR, ZR)])
        if stage_src:
            pltpu.sync_copy(src_in.at[pl.ds(wid * CPT, CPT)], src_r[0])
        plsc.subcore_barrier()

        def body(i, carry):
            q0 = i * CB
            idx = []
            for t in range(CB):
                base = wid * EPT + (q0 + t) * K
                dd = pltpu.async_copy(
                    dstf.at[pl.ds(base, K)], dst_c[t], sem_d[t])
                sl = None
                if not stage_src:
                    sl = pltpu.async_copy(
                        src_in.at[pl.ds(base, K)], src_r[t], sem_i[t])
                idx.append((dd, sl))
            g = [None] * CB
            s = [None] * CB
            for t in range(CB):
                if t >= 2:
                    s[t - 2].wait()
                if stage_src:
                    sidx = src_r[0].at[q0 + t]
                else:
                    idx[t][1].wait()
                    sidx = src_r[t]
                g[t] = pltpu.async_copy(table.at[sidx], rows[t % 2],
                                        sem_g[t % 2])
                if t >= 1:
                    g[t - 1].wait()
                    idx[t - 1][0].wait()
                    s[t - 1] = pltpu.async_copy(
                        rows[(t - 1) % 2], acc.at[dst_c[t - 1]],
                        sem_s[(t - 1) % 2], add=True)
            g[CB - 1].wait()
            idx[CB - 1][0].wait()
            s[CB - 1] = pltpu.async_copy(
                rows[(CB - 1) % 2], acc.at[dst_c[CB - 1]],
                sem_s[(CB - 1) % 2], add=True)
            s[CB - 2].wait()
            s[CB - 1].wait()
            return carry

        lax.fori_loop(0, CPT // CB, body, 0)
        plsc.subcore_barrier()
        pltpu.sync_copy(acc.at[pl.ds(sid * ZR, ZR)],
                        out.at[cid, pl.ds(sid * ZR, ZR)])

    return agg


_SC_AGG_CACHE = {}


def _sc_agg(C, CB, stage_src):
    key = (C, CB, stage_src)
    if key not in _SC_AGG_CACHE:
        _SC_AGG_CACHE[key] = _make_sc_agg(C, CB, stage_src)
    return _SC_AGG_CACHE[key]


def _l0_body(p_ref, x_ref, mwl, mwr, vwl, vwr, mb, vb, h_out, inv_out):
    p = (p_ref[0] + p_ref[1])[:N]
    inv = 1.0 / jnp.maximum(p[:, IN_CH:IN_CH + 1], 1.0)
    agg = p[:, :IN_CH] * inv
    x = x_ref[...]
    m = jnp.maximum(agg @ mwl[...] + mb[...] + x @ mwr[...], 0.0)
    v = jnp.maximum(agg @ vwl[...] + vb[...] + x @ vwr[...], 0.0)
    h_out[...] = jnp.concatenate([m, v], axis=1)
    inv_out[...] = inv


def _layer_body(p_ref, h_ref, inv_ref, mwl, mwr, vwl, vwr, mb, vb, h_out):
    p = (p_ref[0] + p_ref[1])[:N]
    inv = inv_ref[...]
    h = h_ref[...]
    m = jnp.maximum(p[:, :HID] * inv @ mwl[...] + mb[...]
                    + h[:, :HID] @ mwr[...], 0.0)
    v = jnp.maximum(p[:, HID:] * inv @ vwl[...] + vb[...]
                    + h[:, HID:] @ vwr[...], 0.0)
    h_out[...] = jnp.concatenate([m, v], axis=1)


def _head_body(h_ref, eps_ref, b_ref, f1w, f1b, f2w, f2b, out_ref):
    h = h_ref[...]
    z = h[:, :HID] + eps_ref[...] * jnp.exp(0.5 * h[:, HID:])
    gid = lax.broadcasted_iota(jnp.int32, (N, G), 1)
    oh = (b_ref[...] == gid).astype(jnp.float32)
    zsum = lax.dot_general(oh, z, (((0,), (0,)), ((), ())))
    cnt = jnp.sum(oh, axis=0)[:, None]
    zp = zsum / jnp.maximum(cnt, 1.0)
    a = jnp.maximum(zp @ f1w[...] + f1b[...], 0.0)
    logits = a @ f2w[...] + f2b[...]
    mx = jnp.max(logits, axis=1, keepdims=True)
    lse = mx + jnp.log(jnp.sum(jnp.exp(logits - mx), axis=1, keepdims=True))
    out_ref[...] = logits - lse


_EPS_CACHE = None


def _eps():
    global _EPS_CACHE
    if _EPS_CACHE is None:
        _EPS_CACHE = jax.random.normal(
            jax.random.key(42), (N, HID), jnp.float32)
    return _EPS_CACHE


def kernel(x, edge_index, edge_attr, batch, params):
    f32 = jnp.float32
    src = edge_index[0]
    dst = edge_index[1]
    srcf = jnp.concatenate([src, jnp.zeros((EP - E,), jnp.int32)])
    srcp = srcf.reshape(EP // K, K)
    dstf = jnp.concatenate([dst, jnp.full((EP - E,), N, jnp.int32)])
    z144 = jnp.zeros((ZR, IN_CH + 16), f32)
    z128 = jnp.zeros((ZR, 2 * HID), f32)

    table0 = jnp.concatenate(
        [x, jnp.ones((N, 1), f32), jnp.zeros((N, 15), f32)], axis=1)
    p0 = _sc_agg(IN_CH + 16, 5, False)(table0, srcf, z144, dstf)

    b1 = lambda name: params[name].reshape(1, HID)
    h1, inv = pl.pallas_call(
        _l0_body,
        out_shape=(jax.ShapeDtypeStruct((N, 2 * HID), f32),
                   jax.ShapeDtypeStruct((N, 1), f32)),
    )(p0, x, params['mW_l0'], params['mW_r0'], params['vW_l0'],
      params['vW_r0'], b1('mb_l0'), b1('vb_l0'))

    h = h1
    for i in (1, 2):
        p = _sc_agg(2 * HID, 8, True)(h, srcp, z128, dstf)
        h = pl.pallas_call(
            _layer_body,
            out_shape=jax.ShapeDtypeStruct((N, 2 * HID), f32),
        )(p, h, inv, params[f'mW_l{i}'], params[f'mW_r{i}'],
          params[f'vW_l{i}'], params[f'vW_r{i}'],
          b1(f'mb_l{i}'), b1(f'vb_l{i}'))

    logp = pl.pallas_call(
        _head_body,
        out_shape=jax.ShapeDtypeStruct((G, NCLS), f32),
    )(h, _eps(), batch.reshape(N, 1), params['fc1_W'],
      params['fc1_b'].reshape(1, FC_W), params['fc2_W'],
      params['fc2_b'].reshape(1, NCLS))

    return (logp, h[:, :HID], h[:, HID:])
